# Initial kernel scaffold; baseline (speedup 1.0000x reference)
import jax
import jax.numpy as jnp
from jax.experimental import pallas as pl


def kernel(x, edge_index, edge_attr, batch, W1, b1, W2, b2, W3, b3, W4, b4, V1, c1, V2, c2, V3, c3, V4, c4, Wl, bl):
    raise NotImplementedError("write your pallas kernel here")


# speedup vs baseline: 2.5820x; 2.5820x over previous
import functools

import jax
import jax.numpy as jnp
from jax import lax
from jax.experimental import pallas as pl
from jax.experimental.pallas import tpu as pltpu
from jax.experimental.pallas import tpu_sc as plsc

NC, NS = 2, 16
NW = NC * NS


def _sc_mesh():
    return plsc.VectorSubcoreMesh(
        core_axis_name="c", subcore_axis_name="s", num_cores=NC, num_subcores=NS
    )


def _make_gather(E, N, D, CH):
    EW = E // NW
    n_chunks = EW // CH

    @functools.partial(
        pl.kernel,
        out_type=(
            jax.ShapeDtypeStruct((E, D), jnp.float32),
            jax.ShapeDtypeStruct((E, D), jnp.float32),
        ),
        mesh=_sc_mesh(),
        scratch_types=[
            pltpu.VMEM((EW,), jnp.int32),
            pltpu.VMEM((EW,), jnp.int32),
            pltpu.VMEM((CH, D), jnp.float32),
            pltpu.VMEM((CH, D), jnp.float32),
            pltpu.SemaphoreType.DMA,
            pltpu.SemaphoreType.DMA,
        ],
    )
    def gather_k(x_hbm, dst_hbm, src_hbm, xi_hbm, xj_hbm,
                 idx_i, idx_j, rows_i, rows_j, sem_i, sem_j):
        wid = lax.axis_index("s") * NC + lax.axis_index("c")
        base = wid * EW
        pltpu.sync_copy(dst_hbm.at[pl.ds(base, EW)], idx_i)
        pltpu.sync_copy(src_hbm.at[pl.ds(base, EW)], idx_j)

        def body(ci, _):
            off = ci * CH
            cp_i = pltpu.async_copy(
                x_hbm.at[idx_i.at[pl.ds(off, CH)]], rows_i, sem_i)
            cp_j = pltpu.async_copy(
                x_hbm.at[idx_j.at[pl.ds(off, CH)]], rows_j, sem_j)
            cp_i.wait()
            pltpu.sync_copy(rows_i, xi_hbm.at[pl.ds(base + off, CH)])
            cp_j.wait()
            pltpu.sync_copy(rows_j, xj_hbm.at[pl.ds(base + off, CH)])
            return _

        lax.fori_loop(0, n_chunks, body, None)

    return gather_k


def _make_scatter(E, N, D, CH):
    EW = E // NW
    n_chunks = EW // CH

    @functools.partial(
        pl.kernel,
        out_type=jax.ShapeDtypeStruct((NC, N, D), jnp.float32),
        mesh=_sc_mesh(),
        scratch_types=[
            pltpu.VMEM((EW,), jnp.int32),
            pltpu.VMEM((CH, D), jnp.float32),
            pltpu.VMEM_SHARED((N, D), jnp.float32),
            pltpu.SemaphoreType.DMA,
        ],
    )
    def scatter_k(msg_hbm, dst_hbm, zeros_hbm, out_hbm,
                  idx_v, rows_v, aggr_sh, sem):
        c = lax.axis_index("c")
        s = lax.axis_index("s")
        wid = s * NC + c
        base = wid * EW

        @pl.when(s == 0)
        def _():
            pltpu.sync_copy(zeros_hbm, aggr_sh)

        plsc.subcore_barrier()
        pltpu.sync_copy(dst_hbm.at[pl.ds(base, EW)], idx_v)

        def body(ci, _):
            off = ci * CH
            pltpu.sync_copy(msg_hbm.at[pl.ds(base + off, CH)], rows_v)
            pltpu.sync_copy(rows_v, aggr_sh.at[idx_v.at[pl.ds(off, CH)]],
                            add=True)
            return _

        lax.fori_loop(0, n_chunks, body, None)
        plsc.subcore_barrier()

        @pl.when(s == 0)
        def _():
            pltpu.sync_copy(aggr_sh, out_hbm.at[c])

    return scatter_k


def _edge_mlp_body(xi_ref, xj_ref, ea_ref,
                   w1a_ref, w1b_ref, w1c_ref, b1_ref,
                   w2_ref, b2_ref, w3_ref, b3_ref, w4_ref, b4_ref,
                   out_ref):
    h = (jnp.dot(xi_ref[...], w1a_ref[...], preferred_element_type=jnp.float32)
         + jnp.dot(xj_ref[...], w1b_ref[...], preferred_element_type=jnp.float32)
         + jnp.dot(ea_ref[...], w1c_ref[...], preferred_element_type=jnp.float32)
         + b1_ref[...])
    h = jnp.maximum(h, 0.0)
    h = jnp.maximum(
        jnp.dot(h, w2_ref[...], preferred_element_type=jnp.float32) + b2_ref[...], 0.0)
    h = jnp.maximum(
        jnp.dot(h, w3_ref[...], preferred_element_type=jnp.float32) + b3_ref[...], 0.0)
    out_ref[...] = (
        jnp.dot(h, w4_ref[...], preferred_element_type=jnp.float32) + b4_ref[...])


def _run_edge_mlp(xi, xj, ea, w1a, w1b, w1c, b1, w2, b2, w3, b3, w4, b4, BE):
    E, D = xi.shape
    DE = ea.shape[1]
    H = w2.shape[0]
    M = w4.shape[1]
    nblk = E // BE
    full = lambda shape: pl.BlockSpec(shape, lambda i: (0,) * len(shape))
    return pl.pallas_call(
        _edge_mlp_body,
        grid=(nblk,),
        in_specs=[
            pl.BlockSpec((BE, D), lambda i: (i, 0)),
            pl.BlockSpec((BE, D), lambda i: (i, 0)),
            pl.BlockSpec((BE, DE), lambda i: (i, 0)),
            full((D, H)), full((D, H)), full((DE, H)), full((1, H)),
            full((H, H)), full((1, H)),
            full((H, H)), full((1, H)),
            full((H, M)), full((1, M)),
        ],
        out_specs=pl.BlockSpec((BE, M), lambda i: (i, 0)),
        out_shape=jax.ShapeDtypeStruct((E, M), jnp.float32),
    )(xi, xj, ea, w1a, w1b, w1c, b1, w2, b2, w3, b3, w4, b4)


def _node_pool_body(x_ref, ap_ref, batch_ref,
                    v1a_ref, v1b_ref, c1_ref, v2_ref, c2_ref,
                    v3_ref, c3_ref, v4_ref, c4_ref, wl_ref, bl_ref,
                    out_ref, sum_acc, cnt_acc, *, nblk, n_graphs):
    i = pl.program_id(0)

    @pl.when(i == 0)
    def _():
        sum_acc[...] = jnp.zeros_like(sum_acc)
        cnt_acc[...] = jnp.zeros_like(cnt_acc)

    aggr = ap_ref[0] + ap_ref[1]
    h = (jnp.dot(x_ref[...], v1a_ref[...], preferred_element_type=jnp.float32)
         + jnp.dot(aggr, v1b_ref[...], preferred_element_type=jnp.float32)
         + c1_ref[...])
    h = jnp.maximum(h, 0.0)
    h = jnp.maximum(
        jnp.dot(h, v2_ref[...], preferred_element_type=jnp.float32) + c2_ref[...], 0.0)
    h = jnp.maximum(
        jnp.dot(h, v3_ref[...], preferred_element_type=jnp.float32) + c3_ref[...], 0.0)
    node = (jnp.dot(h, v4_ref[...], preferred_element_type=jnp.float32)
            + c4_ref[...])

    b = batch_ref[0]
    gids = lax.broadcasted_iota(jnp.int32, (n_graphs, b.shape[1]), 0)
    mask = (gids == b).astype(jnp.float32)
    sum_acc[...] += jnp.dot(mask, node, preferred_element_type=jnp.float32)
    cnt_acc[...] += jnp.sum(mask, axis=1, keepdims=True)

    @pl.when(i == nblk - 1)
    def _():
        pooled = sum_acc[...] / jnp.maximum(cnt_acc[...], 1.0)
        out_ref[...] = (
            jnp.dot(pooled, wl_ref[...], preferred_element_type=jnp.float32)
            + bl_ref[...])


def _run_node_pool(x, aggr_p, batch3, v1a, v1b, c1, v2, c2, v3, c3, v4, c4,
                   wl, bl, BN, n_graphs):
    N, D = x.shape
    H = v2.shape[0]
    NH = v4.shape[1]
    P = wl.shape[1]
    nblk = N // BN
    full = lambda shape: pl.BlockSpec(shape, lambda i: (0,) * len(shape))
    body = functools.partial(_node_pool_body, nblk=nblk, n_graphs=n_graphs)
    return pl.pallas_call(
        body,
        grid=(nblk,),
        in_specs=[
            pl.BlockSpec((BN, D), lambda i: (i, 0)),
            pl.BlockSpec((NC, BN, D), lambda i: (0, i, 0)),
            pl.BlockSpec((1, 1, BN), lambda i: (i, 0, 0)),
            full((D, H)), full((D, H)), full((1, H)),
            full((H, H)), full((1, H)),
            full((H, H)), full((1, H)),
            full((H, NH)), full((1, NH)),
            full((NH, P)), full((1, P)),
        ],
        out_specs=pl.BlockSpec((n_graphs, P), lambda i: (0, 0)),
        out_shape=jax.ShapeDtypeStruct((n_graphs, P), jnp.float32),
        scratch_shapes=[
            pltpu.VMEM((n_graphs, NH), jnp.float32),
            pltpu.VMEM((n_graphs, 1), jnp.float32),
        ],
    )(x, aggr_p, batch3, v1a, v1b, c1, v2, c2, v3, c3, v4, c4, wl, bl)


def kernel(x, edge_index, edge_attr, batch,
           W1, b1, W2, b2, W3, b3, W4, b4,
           V1, c1, V2, c2, V3, c3, V4, c4,
           Wl, bl):
    N, D = x.shape
    E = edge_index.shape[1]
    DE = edge_attr.shape[1]
    N_GRAPHS = 64
    CH = 80
    BE = 1280
    BN = 1000

    src = edge_index[0]
    dst = edge_index[1]

    xi, xj = _make_gather(E, N, D, CH)(x, dst, src)

    w1a, w1b, w1c = W1[:D], W1[D:2 * D], W1[2 * D:]
    msg = _run_edge_mlp(
        xi, xj, edge_attr,
        w1a, w1b, w1c, b1.reshape(1, -1),
        W2, b2.reshape(1, -1), W3, b3.reshape(1, -1), W4, b4.reshape(1, -1),
        BE)

    zeros = jnp.zeros((N, D), jnp.float32)
    aggr_p = _make_scatter(E, N, D, CH)(msg, dst, zeros)

    batch3 = batch.reshape(N // BN, 1, BN)
    v1a, v1b = V1[:D], V1[D:]
    out = _run_node_pool(
        x, aggr_p, batch3,
        v1a, v1b, c1.reshape(1, -1),
        V2, c2.reshape(1, -1), V3, c3.reshape(1, -1), V4, c4.reshape(1, -1),
        Wl, bl.reshape(1, -1),
        BN, N_GRAPHS)
    return out
